# baseline (speedup 1.0000x reference)
import jax
import jax.numpy as jnp
from jax.experimental import pallas as pl
from jax.experimental.pallas import tpu as pltpu

_CS = 128
_N = 16


def _body(c_hbm, t_hbm, o_hbm, in_buf, out_buf, t_vmem, in_sem, out_sem, t_sem):
    rows = c_hbm.shape[0]
    seq = t_hbm.shape[0]
    total = rows // _CS
    chunks_per_seq = seq // _CS

    def in_cp(i, slot):
        return pltpu.make_async_copy(
            c_hbm.at[pl.ds(i * _CS, _CS), :], in_buf.at[slot], in_sem.at[slot]
        )

    def out_cp(i, slot):
        return pltpu.make_async_copy(
            out_buf.at[slot], o_hbm.at[pl.ds(i * _CS, _CS), :], out_sem.at[slot]
        )

    pltpu.make_async_copy(t_hbm, t_vmem, t_sem).start()
    for j in range(min(_N, total)):
        in_cp(j, j).start()
    pltpu.make_async_copy(t_hbm, t_vmem, t_sem).wait()

    for i in range(total):
        slot = i % _N
        in_cp(i, slot).wait()
        if i >= _N:
            out_cp(i - _N, slot).wait()
        toff = (i % chunks_per_seq) * _CS
        out_buf[slot] = in_buf[slot] + t_vmem[pl.ds(toff, _CS), :]
        out_cp(i, slot).start()
        if i + _N < total:
            in_cp(i + _N, slot).start()

    for i in range(max(total - _N, 0), total):
        out_cp(i, i % _N).wait()


def kernel(concat_embeddings, position_table):
    batch, seq, hidden = concat_embeddings.shape
    flat = concat_embeddings.reshape(batch * seq, hidden)
    table = position_table[:seq]
    out = pl.pallas_call(
        _body,
        in_specs=[
            pl.BlockSpec(memory_space=pl.ANY),
            pl.BlockSpec(memory_space=pl.ANY),
        ],
        out_specs=pl.BlockSpec(memory_space=pl.ANY),
        out_shape=jax.ShapeDtypeStruct((batch * seq, hidden), concat_embeddings.dtype),
        scratch_shapes=[
            pltpu.VMEM((_N, _CS, hidden), concat_embeddings.dtype),
            pltpu.VMEM((_N, _CS, hidden), concat_embeddings.dtype),
            pltpu.VMEM((seq, hidden), concat_embeddings.dtype),
            pltpu.SemaphoreType.DMA((_N,)),
            pltpu.SemaphoreType.DMA((_N,)),
            pltpu.SemaphoreType.DMA,
        ],
    )(flat, table)
    return out.reshape(batch, seq, hidden)


# speedup vs baseline: 1.0306x; 1.0306x over previous
import jax
import jax.numpy as jnp
from jax.experimental import pallas as pl
from jax.experimental.pallas import tpu as pltpu

_CS = 256
_N = 24


def _body(c_hbm, t_hbm, o_hbm, in_buf, out_buf, t_vmem, in_sem, out_sem, t_sem):
    rows = c_hbm.shape[0]
    seq = t_hbm.shape[0]
    total = rows // _CS
    chunks_per_seq = seq // _CS

    def in_cp(i, slot):
        return pltpu.make_async_copy(
            c_hbm.at[pl.ds(i * _CS, _CS), :], in_buf.at[slot], in_sem.at[slot]
        )

    def out_cp(i, slot):
        return pltpu.make_async_copy(
            out_buf.at[slot], o_hbm.at[pl.ds(i * _CS, _CS), :], out_sem.at[slot]
        )

    pltpu.make_async_copy(t_hbm, t_vmem, t_sem).start()
    for j in range(min(_N, total)):
        in_cp(j, j).start()
    pltpu.make_async_copy(t_hbm, t_vmem, t_sem).wait()

    for i in range(total):
        slot = i % _N
        in_cp(i, slot).wait()
        if i >= _N:
            out_cp(i - _N, slot).wait()
        toff = (i % chunks_per_seq) * _CS
        out_buf[slot] = in_buf[slot] + t_vmem[pl.ds(toff, _CS), :]
        out_cp(i, slot).start()
        if i + _N < total:
            in_cp(i + _N, slot).start()

    for i in range(max(total - _N, 0), total):
        out_cp(i, i % _N).wait()


def kernel(concat_embeddings, position_table):
    batch, seq, hidden = concat_embeddings.shape
    flat = concat_embeddings.reshape(batch * seq, hidden)
    table = position_table[:seq]
    out = pl.pallas_call(
        _body,
        in_specs=[
            pl.BlockSpec(memory_space=pl.ANY),
            pl.BlockSpec(memory_space=pl.ANY),
        ],
        out_specs=pl.BlockSpec(memory_space=pl.ANY),
        out_shape=jax.ShapeDtypeStruct((batch * seq, hidden), concat_embeddings.dtype),
        scratch_shapes=[
            pltpu.VMEM((_N, _CS, hidden), concat_embeddings.dtype),
            pltpu.VMEM((_N, _CS, hidden), concat_embeddings.dtype),
            pltpu.VMEM((seq, hidden), concat_embeddings.dtype),
            pltpu.SemaphoreType.DMA((_N,)),
            pltpu.SemaphoreType.DMA((_N,)),
            pltpu.SemaphoreType.DMA,
        ],
    )(flat, table)
    return out.reshape(batch, seq, hidden)
